# baseline (speedup 1.0000x reference)
import functools

import jax
import jax.numpy as jnp
from jax import lax
from jax.experimental import pallas as pl
from jax.experimental.pallas import tpu as pltpu
from jax.experimental.pallas import tpu_sc as plsc

EMB = 64
SCALE = 8.0

B = 4096 * 200
NC = 2
NS = 16
NW = NC * NS
PER_W = B // NW
C = 512
NCHUNK = PER_W // C
G = 128
NG = C // G


def _emb_body(tok_hbm, tab_hbm, out_hbm, idx_v, rows_v, sem):
    wid = lax.axis_index("s") * NC + lax.axis_index("c")
    base_w = wid * PER_W

    def chunk_body(ci, carry):
        base = base_w + ci * C
        pltpu.sync_copy(tok_hbm.at[pl.ds(base, C)], idx_v)
        copies = [
            pltpu.async_copy(
                tab_hbm.at[idx_v.at[pl.ds(j * G, G)]],
                rows_v.at[pl.ds(j * G, G), :],
                sem,
            )
            for j in range(NG)
        ]
        for cp in copies:
            cp.wait()

        def scale_body(r, c2):
            for c4 in range(EMB // 16):
                sl = pl.ds(c4 * 16, 16)
                rows_v[r, sl] = rows_v[r, sl] * SCALE
            return c2

        lax.fori_loop(0, C, scale_body, 0)
        pltpu.sync_copy(rows_v, out_hbm.at[pl.ds(base, C)])
        return carry

    lax.fori_loop(0, NCHUNK, chunk_body, 0)


@functools.partial(
    pl.kernel,
    mesh=plsc.VectorSubcoreMesh(core_axis_name="c", subcore_axis_name="s"),
    out_type=jax.ShapeDtypeStruct((B, EMB), jnp.float32),
    scratch_types=[
        pltpu.VMEM((C,), jnp.int32),
        pltpu.VMEM((C, EMB), jnp.float32),
        pltpu.SemaphoreType.DMA,
    ],
    compiler_params=pltpu.CompilerParams(use_tc_tiling_on_sc=False),
)
def _emb_kernel(tok_hbm, tab_hbm, out_hbm, idx_v, rows_v, sem):
    _emb_body(tok_hbm, tab_hbm, out_hbm, idx_v, rows_v, sem)


def kernel(tokens, table):
    tok_flat = tokens.reshape(-1).astype(jnp.int32)
    out = _emb_kernel(tok_flat, table)
    return out.reshape(tokens.shape + (EMB,))


# speedup vs baseline: 1.1224x; 1.1224x over previous
import functools

import jax
import jax.numpy as jnp
from jax import lax
from jax.experimental import pallas as pl
from jax.experimental.pallas import tpu as pltpu
from jax.experimental.pallas import tpu_sc as plsc

EMB = 64
SCALE = 8.0

B = 4096 * 200
NC = 2
NS = 16
NW = NC * NS
PER_W = B // NW
C = 512
NCHUNK = PER_W // C
G = 128
NG = C // G


def _emb_body(tok_hbm, tab_hbm, out_hbm,
              idx0, idx1, rows0, rows1, gsem0, gsem1, wsem0, wsem1):
    wid = lax.axis_index("s") * NC + lax.axis_index("c")
    base_w = wid * PER_W
    idx = (idx0, idx1)
    rows = (rows0, rows1)
    gsem = (gsem0, gsem1)
    wsem = (wsem0, wsem1)

    def fire_gather(ci, b):
        base = base_w + ci * C
        pltpu.sync_copy(tok_hbm.at[pl.ds(base, C)], idx[b])
        for j in range(NG):
            pltpu.async_copy(
                tab_hbm.at[idx[b].at[pl.ds(j * G, G)]],
                rows[b].at[pl.ds(j * G, G), :],
                gsem[b],
            )

    def drain_gather(b):
        pltpu.make_async_copy(tab_hbm.at[pl.ds(0, C)], rows[b], gsem[b]).wait()

    def drain_write(b, ci):
        pltpu.make_async_copy(
            rows[b], out_hbm.at[pl.ds(base_w + ci * C, C)], wsem[b]).wait()

    def scale(b):
        rb = rows[b]

        @plsc.parallel_loop(0, C, unroll=4)
        def _(r):
            for c4 in range(EMB // 16):
                sl = pl.ds(c4 * 16, 16)
                rb[r, sl] = rb[r, sl] * SCALE

    fire_gather(0, 0)

    def outer(co, carry):
        for b in range(2):
            ci = co * 2 + b

            @pl.when(ci >= 1)
            def _():
                drain_write(1 - b, ci - 1)

            @pl.when(ci + 1 < NCHUNK)
            def _():
                fire_gather(ci + 1, 1 - b)

            drain_gather(b)
            scale(b)
            pltpu.async_copy(
                rows[b], out_hbm.at[pl.ds(base_w + ci * C, C)], wsem[b])
        return carry

    lax.fori_loop(0, NCHUNK // 2, outer, 0)
    drain_write(1, NCHUNK - 1)


@functools.partial(
    pl.kernel,
    mesh=plsc.VectorSubcoreMesh(core_axis_name="c", subcore_axis_name="s"),
    out_type=jax.ShapeDtypeStruct((B, EMB), jnp.float32),
    scratch_types=[
        pltpu.VMEM((C,), jnp.int32),
        pltpu.VMEM((C,), jnp.int32),
        pltpu.VMEM((C, EMB), jnp.float32),
        pltpu.VMEM((C, EMB), jnp.float32),
        pltpu.SemaphoreType.DMA,
        pltpu.SemaphoreType.DMA,
        pltpu.SemaphoreType.DMA,
        pltpu.SemaphoreType.DMA,
    ],
    compiler_params=pltpu.CompilerParams(use_tc_tiling_on_sc=False),
)
def _emb_kernel(tok_hbm, tab_hbm, out_hbm,
                idx0, idx1, rows0, rows1, gsem0, gsem1, wsem0, wsem1):
    _emb_body(tok_hbm, tab_hbm, out_hbm,
              idx0, idx1, rows0, rows1, gsem0, gsem1, wsem0, wsem1)


def kernel(tokens, table):
    tok_flat = tokens.reshape(-1).astype(jnp.int32)
    out = _emb_kernel(tok_flat, table)
    return out.reshape(tokens.shape + (EMB,))
